# baseline (speedup 1.0000x reference)
import functools

import jax
import jax.numpy as jnp
from jax import lax
from jax.experimental import pallas as pl
from jax.experimental.pallas import tpu as pltpu
from jax.experimental.pallas import tpu_sc as plsc

N = 10000
E = 160000
HID = 256
NH = HID // 2
CH = 128
NC = 2
NS = 16
NPAD = N + 8
CPS = 80
NHALF = 2
HCH = CPS // NHALF
EPAD = NS * CPS * CH
DEGC = CPS // NC
ROWS_PER_SUB = 624
ROWS_TAIL = NPAD - NS * ROWS_PER_SUB

RB = 2000
GRID = N // RB


def _sc_mesh():
    return plsc.VectorSubcoreMesh(
        core_axis_name="c", subcore_axis_name="s", num_cores=NC, num_subcores=NS)


def _zero_acc(z_hbm, acc_sh, s):
    rows = pl.ds(s * ROWS_PER_SUB, ROWS_PER_SUB)
    pltpu.sync_copy(z_hbm.at[rows], acc_sh.at[rows])

    @pl.when(s == NS - 1)
    def _():
        tail = pl.ds(NS * ROWS_PER_SUB, ROWS_TAIL)
        pltpu.sync_copy(z_hbm.at[tail], acc_sh.at[tail])


def _writeout(acc_sh, out_hbm, c, s):
    base = pl.multiple_of(c * NPAD, 8)
    rows = pl.ds(s * ROWS_PER_SUB, ROWS_PER_SUB)
    orows = pl.ds(base + s * ROWS_PER_SUB, ROWS_PER_SUB)
    pltpu.sync_copy(acc_sh.at[rows], out_hbm.at[orows])

    @pl.when(s == NS - 1)
    def _():
        tail = pl.ds(NS * ROWS_PER_SUB, ROWS_TAIL)
        otail = pl.ds(base + NS * ROWS_PER_SUB, ROWS_TAIL)
        pltpu.sync_copy(acc_sh.at[tail], out_hbm.at[otail])


def _sc_degree(src3dw, zeros_nh, ones_nh):
    @functools.partial(
        pl.kernel,
        out_type=jax.ShapeDtypeStruct((NC * NPAD, NH), jnp.float32),
        mesh=_sc_mesh(),
        scratch_types=[
            pltpu.VMEM((DEGC, CH), jnp.int32),
            pltpu.VMEM((CH, NH), jnp.float32),
            pltpu.VMEM_SHARED((NPAD, NH), jnp.float32),
            pltpu.SemaphoreType.DMA,
        ],
    )
    def deg_kernel(src_hbm, z_hbm, ones_hbm, out_hbm, si_d, ones_v, acc_sh, sem):
        c = lax.axis_index("c")
        s = lax.axis_index("s")
        _zero_acc(z_hbm, acc_sh, s)
        pltpu.sync_copy(src_hbm.at[s, pl.ds(c * DEGC, DEGC)], si_d)
        pltpu.sync_copy(ones_hbm, ones_v)
        plsc.subcore_barrier()

        @pl.loop(0, DEGC)
        def _(k):
            pltpu.async_copy(ones_v, acc_sh.at[si_d.at[k]], sem, add=True)

        @pl.loop(0, DEGC)
        def _(k):
            pltpu.make_async_copy(ones_v, acc_sh.at[si_d.at[0]], sem).wait()

        plsc.subcore_barrier()
        _writeout(acc_sh, out_hbm, c, s)

    return deg_kernel(src3dw, zeros_nh, ones_nh)


def _sc_segsum(tab, src3dw, dst3d, zeros_nh):
    @functools.partial(
        pl.kernel,
        out_type=jax.ShapeDtypeStruct((NC * NPAD, NH), jnp.float32),
        mesh=_sc_mesh(),
        scratch_types=[
            pltpu.VMEM((HCH, CH), jnp.int32),
            pltpu.VMEM((HCH, CH), jnp.int32),
            pltpu.VMEM((CH, NH), jnp.float32),
            pltpu.VMEM((CH, NH), jnp.float32),
            pltpu.VMEM_SHARED((NPAD, NH), jnp.float32),
            pltpu.SemaphoreType.DMA,
            pltpu.SemaphoreType.DMA,
            pltpu.SemaphoreType.DMA,
            pltpu.SemaphoreType.DMA,
        ],
    )
    def seg_kernel(tab_hbm, src_hbm, dst_hbm, z_hbm, out_hbm,
                   si_v, di_v, buf0, buf1, acc_sh, g0, g1, s0, s1):
        c = lax.axis_index("c")
        s = lax.axis_index("s")
        w = c * NS + s
        bufs = (buf0, buf1)
        gsem = (g0, g1)
        ssem = (s0, s1)
        _zero_acc(z_hbm, acc_sh, s)
        plsc.subcore_barrier()

        def fire(k, b):
            pltpu.async_copy(tab_hbm.at[si_v.at[k]], bufs[b], gsem[b])

        def wait_g(k, b):
            pltpu.make_async_copy(tab_hbm.at[si_v.at[k]], bufs[b], gsem[b]).wait()

        def scat(k, b):
            pltpu.async_copy(bufs[b], acc_sh.at[di_v.at[k]], ssem[b], add=True)

        def wait_s(k, b):
            pltpu.make_async_copy(bufs[b], acc_sh.at[di_v.at[k]], ssem[b]).wait()

        @pl.loop(0, NHALF)
        def _(hh):
            off = pl.multiple_of(hh * HCH, 8)
            pltpu.sync_copy(src_hbm.at[w, pl.ds(off, HCH)], si_v)
            pltpu.sync_copy(dst_hbm.at[s, pl.ds(off, HCH)], di_v)
            fire(0, 0)
            fire(1, 1)

            @pl.loop(0, HCH, step=2)
            def _(j):
                wait_g(j, 0)
                scat(j, 0)
                wait_g(j + 1, 1)
                scat(j + 1, 1)
                wait_s(j, 0)

                @pl.when(j < HCH - 2)
                def _():
                    fire(j + 2, 0)

                wait_s(j + 1, 1)

                @pl.when(j < HCH - 2)
                def _():
                    fire(j + 3, 1)

        plsc.subcore_barrier()
        _writeout(acc_sh, out_hbm, c, s)

    return seg_kernel(tab, src3dw, dst3d, zeros_nh)


def _tc_step0(x0, wx, bc):
    def body(x_ref, wx_ref, b_ref, h_ref, c_ref):
        z = x_ref[...] * wx_ref[...] + b_ref[...]
        i = jax.nn.sigmoid(z[:, 0:HID])
        t = jnp.tanh(z[:, 2 * HID:3 * HID])
        o = jax.nn.sigmoid(z[:, 3 * HID:4 * HID])
        cc = i * t
        c_ref[...] = cc
        h_ref[...] = o * jnp.tanh(cc)

    return pl.pallas_call(
        body,
        grid=(GRID,),
        in_specs=[
            pl.BlockSpec((RB, 1), lambda i: (i, 0)),
            pl.BlockSpec((1, 4 * HID), lambda i: (0, 0)),
            pl.BlockSpec((1, 4 * HID), lambda i: (0, 0)),
        ],
        out_specs=[pl.BlockSpec((RB, HID), lambda i: (i, 0))] * 2,
        out_shape=[jax.ShapeDtypeStruct((N, HID), jnp.float32)] * 2,
    )(x0, wx, bc)


def _tc_prep(h, dega, degb):
    def body(h_ref, da_ref, db_ref, dinv_ref, h0_ref, h1_ref):
        deg = da_ref[...][:, 0:1] + db_ref[...][:, 0:1]
        dinv = jnp.where(deg > 0, 1.0 / jnp.sqrt(jnp.maximum(deg, 1e-12)), 0.0)
        dinv_ref[...] = dinv
        hs = h_ref[...] * dinv
        h0_ref[...] = hs[:, 0:NH]
        h1_ref[...] = hs[:, NH:HID]

    return pl.pallas_call(
        body,
        grid=(GRID,),
        in_specs=[
            pl.BlockSpec((RB, HID), lambda i: (i, 0)),
            pl.BlockSpec((RB, NH), lambda i: (i, 0)),
            pl.BlockSpec((RB, NH), lambda i: (i, 0)),
        ],
        out_specs=[
            pl.BlockSpec((RB, 1), lambda i: (i, 0)),
            pl.BlockSpec((RB, NH), lambda i: (i, 0)),
            pl.BlockSpec((RB, NH), lambda i: (i, 0)),
        ],
        out_shape=[
            jax.ShapeDtypeStruct((N, 1), jnp.float32),
            jax.ShapeDtypeStruct((N, NH), jnp.float32),
            jax.ShapeDtypeStruct((N, NH), jnp.float32),
        ],
    )(h, dega, degb)


def _tc_step(xt, h, c, tx0, tx1, dinv, w0, w1a, w1b, wx, bc):
    def body(x_ref, h_ref, c_ref, t0_ref, t1_ref, dv_ref,
             w0_ref, w1a_ref, w1b_ref, wx_ref, b_ref,
             hn_ref, cn_ref, h0_ref, h1_ref):
        dv = dv_ref[...]
        nd = -dv
        z = jnp.dot(h_ref[...], w0_ref[...], preferred_element_type=jnp.float32)
        z = z + jnp.dot(t0_ref[...] * nd, w1a_ref[...],
                        preferred_element_type=jnp.float32)
        z = z + jnp.dot(t1_ref[...] * nd, w1b_ref[...],
                        preferred_element_type=jnp.float32)
        z = z + x_ref[...] * wx_ref[...] + b_ref[...]
        i = jax.nn.sigmoid(z[:, 0:HID])
        f = jax.nn.sigmoid(z[:, HID:2 * HID])
        t = jnp.tanh(z[:, 2 * HID:3 * HID])
        o = jax.nn.sigmoid(z[:, 3 * HID:4 * HID])
        cn = f * c_ref[...] + i * t
        hn = o * jnp.tanh(cn)
        hn_ref[...] = hn
        cn_ref[...] = cn
        hs = hn * dv
        h0_ref[...] = hs[:, 0:NH]
        h1_ref[...] = hs[:, NH:HID]

    return pl.pallas_call(
        body,
        grid=(GRID,),
        in_specs=[
            pl.BlockSpec((RB, 1), lambda i: (i, 0)),
            pl.BlockSpec((RB, HID), lambda i: (i, 0)),
            pl.BlockSpec((RB, HID), lambda i: (i, 0)),
            pl.BlockSpec((RB, NH), lambda i: (i, 0)),
            pl.BlockSpec((RB, NH), lambda i: (i, 0)),
            pl.BlockSpec((RB, 1), lambda i: (i, 0)),
            pl.BlockSpec((HID, 4 * HID), lambda i: (0, 0)),
            pl.BlockSpec((NH, 4 * HID), lambda i: (0, 0)),
            pl.BlockSpec((NH, 4 * HID), lambda i: (0, 0)),
            pl.BlockSpec((1, 4 * HID), lambda i: (0, 0)),
            pl.BlockSpec((1, 4 * HID), lambda i: (0, 0)),
        ],
        out_specs=[
            pl.BlockSpec((RB, HID), lambda i: (i, 0)),
            pl.BlockSpec((RB, HID), lambda i: (i, 0)),
            pl.BlockSpec((RB, NH), lambda i: (i, 0)),
            pl.BlockSpec((RB, NH), lambda i: (i, 0)),
        ],
        out_shape=[
            jax.ShapeDtypeStruct((N, HID), jnp.float32),
            jax.ShapeDtypeStruct((N, HID), jnp.float32),
            jax.ShapeDtypeStruct((N, NH), jnp.float32),
            jax.ShapeDtypeStruct((N, NH), jnp.float32),
        ],
    )(xt, h, c, tx0, tx1, dinv, w0, w1a, w1b, wx, bc)


def kernel(x_seq, edge_index, W_i, b_i, Wch_i, bch_i, W_f, b_f, Wch_f, bch_f,
           W_c, b_c, Wch_c, bch_c, W_o, b_o, Wch_o, bch_o):
    pad = jnp.full((EPAD - E,), N, jnp.int32)
    bsrc = jnp.concatenate([edge_index[0], pad]).reshape(NS, CPS, CH)
    bdst = jnp.concatenate([edge_index[1], pad]).reshape(NS, CPS, CH)
    src3dw = jnp.concatenate([bsrc, bsrc + NPAD], axis=0)

    w0 = jnp.concatenate([Wch_i[0], Wch_f[0], Wch_c[0], Wch_o[0]], axis=1)
    w1 = jnp.concatenate([Wch_i[1], Wch_f[1], Wch_c[1], Wch_o[1]], axis=1)
    w1a = w1[0:NH]
    w1b = w1[NH:HID]
    wx = jnp.concatenate([W_i, W_f, W_c, W_o], axis=1)
    bc = jnp.concatenate([
        b_i + bch_i[None, :], b_f + bch_f[None, :],
        b_c + bch_c[None, :], b_o + bch_o[None, :]], axis=1)

    zeros_nh = jnp.zeros((NPAD, NH), jnp.float32)
    ones_nh = jnp.ones((CH, NH), jnp.float32)
    z8 = jnp.zeros((NPAD - N, NH), jnp.float32)

    degcat = _sc_degree(src3dw, zeros_nh, ones_nh)
    h, cst = _tc_step0(x_seq[0], wx, bc)
    dinv, hs0, hs1 = _tc_prep(h, degcat[:N], degcat[NPAD:NPAD + N])

    for t in range(1, 3):
        tab = jnp.concatenate([hs0, z8, hs1, z8], axis=0)
        tcat = _sc_segsum(tab, src3dw, bdst, zeros_nh)
        h, cst, hs0, hs1 = _tc_step(
            x_seq[t], h, cst, tcat[:N], tcat[NPAD:NPAD + N],
            dinv, w0, w1a, w1b, wx, bc)

    return h


# speedup vs baseline: 1.1507x; 1.1507x over previous
import functools

import jax
import jax.numpy as jnp
from jax import lax
from jax.experimental import pallas as pl
from jax.experimental.pallas import tpu as pltpu
from jax.experimental.pallas import tpu_sc as plsc

N = 10000
E = 160000
HID = 256
NH = HID // 2
CH = 128
NC = 2
NS = 16
NPAD = N + 8
CPS = 80
NHALF = 2
HCH = CPS // NHALF
EPAD = NS * CPS * CH
DEGC = CPS // NC
ROWS_PER_SUB = 624
ROWS_TAIL = NPAD - NS * ROWS_PER_SUB

RB = 2000
GRID = N // RB


def _sc_mesh():
    return plsc.VectorSubcoreMesh(
        core_axis_name="c", subcore_axis_name="s", num_cores=NC, num_subcores=NS)


def _zero_acc(z_hbm, acc_sh, s):
    rows = pl.ds(s * ROWS_PER_SUB, ROWS_PER_SUB)
    pltpu.sync_copy(z_hbm.at[rows], acc_sh.at[rows])

    @pl.when(s == NS - 1)
    def _():
        tail = pl.ds(NS * ROWS_PER_SUB, ROWS_TAIL)
        pltpu.sync_copy(z_hbm.at[tail], acc_sh.at[tail])


def _writeout(acc_sh, out_hbm, c, s):
    rows = pl.ds(s * ROWS_PER_SUB, ROWS_PER_SUB)
    pltpu.sync_copy(acc_sh.at[rows], out_hbm.at[c, rows])

    @pl.when(s == NS - 1)
    def _():
        tail = pl.ds(NS * ROWS_PER_SUB, ROWS_TAIL)
        pltpu.sync_copy(acc_sh.at[tail], out_hbm.at[c, tail])


def _sc_degree(src3dw, zeros_nh, ones_nh):
    @functools.partial(
        pl.kernel,
        out_type=jax.ShapeDtypeStruct((NC, NPAD, NH), jnp.float32),
        mesh=_sc_mesh(),
        scratch_types=[
            pltpu.VMEM((DEGC, CH), jnp.int32),
            pltpu.VMEM((CH, NH), jnp.float32),
            pltpu.VMEM_SHARED((NPAD, NH), jnp.float32),
            pltpu.SemaphoreType.DMA,
        ],
    )
    def deg_kernel(src_hbm, z_hbm, ones_hbm, out_hbm, si_d, ones_v, acc_sh, sem):
        c = lax.axis_index("c")
        s = lax.axis_index("s")
        _zero_acc(z_hbm, acc_sh, s)
        pltpu.sync_copy(src_hbm.at[s, pl.ds(c * DEGC, DEGC)], si_d)
        pltpu.sync_copy(ones_hbm, ones_v)
        plsc.subcore_barrier()

        @pl.loop(0, DEGC)
        def _(k):
            pltpu.async_copy(ones_v, acc_sh.at[si_d.at[k]], sem, add=True)

        @pl.loop(0, DEGC)
        def _(k):
            pltpu.make_async_copy(ones_v, acc_sh.at[si_d.at[0]], sem).wait()

        plsc.subcore_barrier()
        _writeout(acc_sh, out_hbm, c, s)

    return deg_kernel(src3dw, zeros_nh, ones_nh)


def _sc_segsum(hs0p, hs1p, src3d, dst3d, zeros_nh):
    @functools.partial(
        pl.kernel,
        out_type=jax.ShapeDtypeStruct((NC, NPAD, NH), jnp.float32),
        mesh=_sc_mesh(),
        scratch_types=[
            pltpu.VMEM((HCH, CH), jnp.int32),
            pltpu.VMEM((HCH, CH), jnp.int32),
            pltpu.VMEM((CH, NH), jnp.float32),
            pltpu.VMEM((CH, NH), jnp.float32),
            pltpu.VMEM_SHARED((NPAD, NH), jnp.float32),
            pltpu.SemaphoreType.DMA,
            pltpu.SemaphoreType.DMA,
            pltpu.SemaphoreType.DMA,
            pltpu.SemaphoreType.DMA,
        ],
    )
    def seg_kernel(h0_hbm, h1_hbm, src_hbm, dst_hbm, z_hbm, out_hbm,
                   si_v, di_v, buf0, buf1, acc_sh, g0, g1, s0, s1):
        c = lax.axis_index("c")
        s = lax.axis_index("s")
        bufs = (buf0, buf1)
        gsem = (g0, g1)
        ssem = (s0, s1)
        _zero_acc(z_hbm, acc_sh, s)
        plsc.subcore_barrier()

        def fire(k, b):
            @pl.when(c == 0)
            def _():
                pltpu.async_copy(h0_hbm.at[si_v.at[k]], bufs[b], gsem[b])

            @pl.when(c == 1)
            def _():
                pltpu.async_copy(h1_hbm.at[si_v.at[k]], bufs[b], gsem[b])

        def wait_g(k, b):
            pltpu.make_async_copy(h0_hbm.at[si_v.at[k]], bufs[b], gsem[b]).wait()

        def scat(k, b):
            pltpu.async_copy(bufs[b], acc_sh.at[di_v.at[k]], ssem[b], add=True)

        def wait_s(k, b):
            pltpu.make_async_copy(bufs[b], acc_sh.at[di_v.at[k]], ssem[b]).wait()

        @pl.loop(0, NHALF)
        def _(hh):
            off = pl.multiple_of(hh * HCH, 8)
            pltpu.sync_copy(src_hbm.at[s, pl.ds(off, HCH)], si_v)
            pltpu.sync_copy(dst_hbm.at[s, pl.ds(off, HCH)], di_v)
            fire(0, 0)
            fire(1, 1)

            @pl.loop(0, HCH, step=2)
            def _(j):
                wait_g(j, 0)
                scat(j, 0)
                wait_g(j + 1, 1)
                scat(j + 1, 1)
                wait_s(j, 0)

                @pl.when(j < HCH - 2)
                def _():
                    fire(j + 2, 0)

                wait_s(j + 1, 1)

                @pl.when(j < HCH - 2)
                def _():
                    fire(j + 3, 1)

        plsc.subcore_barrier()
        _writeout(acc_sh, out_hbm, c, s)

    return seg_kernel(hs0p, hs1p, src3d, dst3d, zeros_nh)


def _tc_step0(x0, wx, bc):
    def body(x_ref, wx_ref, b_ref, h_ref, c_ref):
        z = x_ref[...] * wx_ref[...] + b_ref[...]
        i = jax.nn.sigmoid(z[:, 0:HID])
        t = jnp.tanh(z[:, 2 * HID:3 * HID])
        o = jax.nn.sigmoid(z[:, 3 * HID:4 * HID])
        cc = i * t
        c_ref[...] = cc
        h_ref[...] = o * jnp.tanh(cc)

    return pl.pallas_call(
        body,
        grid=(GRID,),
        in_specs=[
            pl.BlockSpec((RB, 1), lambda i: (i, 0)),
            pl.BlockSpec((1, 4 * HID), lambda i: (0, 0)),
            pl.BlockSpec((1, 4 * HID), lambda i: (0, 0)),
        ],
        out_specs=[pl.BlockSpec((RB, HID), lambda i: (i, 0))] * 2,
        out_shape=[jax.ShapeDtypeStruct((N, HID), jnp.float32)] * 2,
    )(x0, wx, bc)


def _tc_prep(h, degp):
    def body(h_ref, dp_ref, dinv_ref, h0_ref, h1_ref):
        deg = dp_ref[0][:, 0:1] + dp_ref[1][:, 0:1]
        dinv = jnp.where(deg > 0, 1.0 / jnp.sqrt(jnp.maximum(deg, 1e-12)), 0.0)
        dinv_ref[...] = dinv
        hs = h_ref[...] * dinv
        h0_ref[...] = hs[:, 0:NH]
        h1_ref[...] = hs[:, NH:HID]

    return pl.pallas_call(
        body,
        grid=(GRID,),
        in_specs=[
            pl.BlockSpec((RB, HID), lambda i: (i, 0)),
            pl.BlockSpec((NC, RB, NH), lambda i: (0, i, 0)),
        ],
        out_specs=[
            pl.BlockSpec((RB, 1), lambda i: (i, 0)),
            pl.BlockSpec((RB, NH), lambda i: (i, 0)),
            pl.BlockSpec((RB, NH), lambda i: (i, 0)),
        ],
        out_shape=[
            jax.ShapeDtypeStruct((N, 1), jnp.float32),
            jax.ShapeDtypeStruct((NPAD, NH), jnp.float32),
            jax.ShapeDtypeStruct((NPAD, NH), jnp.float32),
        ],
    )(h, degp)


def _tc_step(xt, h, c, tx, dinv, w0, w1a, w1b, wx, bc):
    def body(x_ref, h_ref, c_ref, t_ref, dv_ref,
             w0_ref, w1a_ref, w1b_ref, wx_ref, b_ref,
             hn_ref, cn_ref, h0_ref, h1_ref):
        dv = dv_ref[...]
        nd = -dv
        z = jnp.dot(h_ref[...], w0_ref[...], preferred_element_type=jnp.float32)
        z = z + jnp.dot(t_ref[0] * nd, w1a_ref[...],
                        preferred_element_type=jnp.float32)
        z = z + jnp.dot(t_ref[1] * nd, w1b_ref[...],
                        preferred_element_type=jnp.float32)
        z = z + x_ref[...] * wx_ref[...] + b_ref[...]
        i = jax.nn.sigmoid(z[:, 0:HID])
        f = jax.nn.sigmoid(z[:, HID:2 * HID])
        t = jnp.tanh(z[:, 2 * HID:3 * HID])
        o = jax.nn.sigmoid(z[:, 3 * HID:4 * HID])
        cn = f * c_ref[...] + i * t
        hn = o * jnp.tanh(cn)
        hn_ref[...] = hn
        cn_ref[...] = cn
        hs = hn * dv
        h0_ref[...] = hs[:, 0:NH]
        h1_ref[...] = hs[:, NH:HID]

    return pl.pallas_call(
        body,
        grid=(GRID,),
        in_specs=[
            pl.BlockSpec((RB, 1), lambda i: (i, 0)),
            pl.BlockSpec((RB, HID), lambda i: (i, 0)),
            pl.BlockSpec((RB, HID), lambda i: (i, 0)),
            pl.BlockSpec((NC, RB, NH), lambda i: (0, i, 0)),
            pl.BlockSpec((RB, 1), lambda i: (i, 0)),
            pl.BlockSpec((HID, 4 * HID), lambda i: (0, 0)),
            pl.BlockSpec((NH, 4 * HID), lambda i: (0, 0)),
            pl.BlockSpec((NH, 4 * HID), lambda i: (0, 0)),
            pl.BlockSpec((1, 4 * HID), lambda i: (0, 0)),
            pl.BlockSpec((1, 4 * HID), lambda i: (0, 0)),
        ],
        out_specs=[
            pl.BlockSpec((RB, HID), lambda i: (i, 0)),
            pl.BlockSpec((RB, HID), lambda i: (i, 0)),
            pl.BlockSpec((RB, NH), lambda i: (i, 0)),
            pl.BlockSpec((RB, NH), lambda i: (i, 0)),
        ],
        out_shape=[
            jax.ShapeDtypeStruct((N, HID), jnp.float32),
            jax.ShapeDtypeStruct((N, HID), jnp.float32),
            jax.ShapeDtypeStruct((NPAD, NH), jnp.float32),
            jax.ShapeDtypeStruct((NPAD, NH), jnp.float32),
        ],
    )(xt, h, c, tx, dinv, w0, w1a, w1b, wx, bc)


def kernel(x_seq, edge_index, W_i, b_i, Wch_i, bch_i, W_f, b_f, Wch_f, bch_f,
           W_c, b_c, Wch_c, bch_c, W_o, b_o, Wch_o, bch_o):
    pad = jnp.full((EPAD - E,), N, jnp.int32)
    bsrc = jnp.concatenate([edge_index[0], pad]).reshape(NS, CPS, CH)
    bdst = jnp.concatenate([edge_index[1], pad]).reshape(NS, CPS, CH)

    w0 = jnp.concatenate([Wch_i[0], Wch_f[0], Wch_c[0], Wch_o[0]], axis=1)
    w1 = jnp.concatenate([Wch_i[1], Wch_f[1], Wch_c[1], Wch_o[1]], axis=1)
    w1a = w1[0:NH]
    w1b = w1[NH:HID]
    wx = jnp.concatenate([W_i, W_f, W_c, W_o], axis=1)
    bc = jnp.concatenate([
        b_i + bch_i[None, :], b_f + bch_f[None, :],
        b_c + bch_c[None, :], b_o + bch_o[None, :]], axis=1)

    zeros_nh = jnp.zeros((NPAD, NH), jnp.float32)
    ones_nh = jnp.ones((CH, NH), jnp.float32)

    degp = _sc_degree(bsrc, zeros_nh, ones_nh)
    h, cst = _tc_step0(x_seq[0], wx, bc)
    dinv, hs0, hs1 = _tc_prep(h, degp)

    for t in range(1, 3):
        tx = _sc_segsum(hs0, hs1, bsrc, bdst, zeros_nh)
        h, cst, hs0, hs1 = _tc_step(
            x_seq[t], h, cst, tx, dinv, w0, w1a, w1b, wx, bc)

    return h
